# baseline (speedup 1.0000x reference)
import jax
import jax.numpy as jnp
from jax import lax
from jax.experimental import pallas as pl
from jax.experimental.pallas import tpu as pltpu
from jax.experimental.pallas import tpu_sc as plsc

N_NODES = 10000
D = 128
DH = D // 2
NC = 2
NS = 16
G = 64
SCH = 8
GPC = SCH * 128 // G
NBUF = 2
PAIR_ROWS = 5120
RPT = PAIR_ROWS // NS
ZCH = 64
LANES = 16
HSL = DH // LANES


def _mm_body(xe_ref, xo_ref, w_ref, o_ref):
    lo = jnp.dot(xe_ref[...], w_ref[0], preferred_element_type=jnp.float32)
    hi = jnp.dot(xo_ref[...], w_ref[0], preferred_element_type=jnp.float32)
    o_ref[...] = jnp.concatenate([lo, hi], axis=-1)[None]


def _matmul(xe, xo, w):
    bm = 512
    grid_m = PAIR_ROWS // bm
    return pl.pallas_call(
        _mm_body,
        grid=(NC, grid_m),
        in_specs=[
            pl.BlockSpec((bm, D), lambda c, i: (i, 0)),
            pl.BlockSpec((bm, D), lambda c, i: (i, 0)),
            pl.BlockSpec((1, D, DH), lambda c, i: (c, 0, 0)),
        ],
        out_specs=pl.BlockSpec((1, bm, D), lambda c, i: (c, i, 0)),
        out_shape=jax.ShapeDtypeStruct((NC, PAIR_ROWS, D), jnp.float32),
    )(xe, xo, w)


def _spmm_body(nchunks):
    def body(support_hbm, src_hbm, dst_hbm, val_hbm, out_hbm,
             src_c, dst_c, val_c, rows_v, srow_v, spair0, spair1, dpair,
             sup_sh, acc_sh, sem0, sem1):
        sems = (sem0, sem1)
        spairs = (spair0, spair1)
        cid = lax.axis_index("c")
        sid = lax.axis_index("s")

        for k in range(RPT // ZCH):
            r0 = sid * RPT + k * ZCH
            pltpu.sync_copy(support_hbm.at[cid, pl.ds(r0, ZCH)],
                            rows_v.at[0])
            pltpu.sync_copy(rows_v.at[0], sup_sh.at[pl.ds(r0, ZCH)])

        zeros16 = jnp.zeros((LANES,), jnp.float32)

        @pl.loop(0, ZCH)
        def _(i):
            for j in range(D // LANES):
                rows_v[0, i, pl.ds(j * LANES, LANES)] = zeros16

        for k in range(RPT // ZCH):
            pltpu.sync_copy(
                rows_v.at[0],
                acc_sh.at[pl.ds(sid * RPT + k * ZCH, ZCH)])
        plsc.subcore_barrier()

        @pl.loop(0, nchunks)
        def _(c):
            pltpu.sync_copy(src_hbm.at[sid, pl.ds(c * SCH, SCH)], src_c)
            pltpu.sync_copy(dst_hbm.at[sid, pl.ds(c * SCH, SCH)], dst_c)
            pltpu.sync_copy(val_hbm.at[sid, pl.ds(c * SCH, SCH)], val_c)

            def start_gather(g, b):
                r, base = g // 2, (g % 2) * G
                pb = spairs[b]

                @pl.loop(0, G // LANES)
                def _(s):
                    pb[pl.ds(s * LANES, LANES)] = lax.shift_right_logical(
                        src_c[r, pl.ds(base + s * LANES, LANES)], 1)

                return pltpu.async_copy(sup_sh.at[pb], rows_v.at[b], sems[b])

            descs = [None] * NBUF
            descs[0] = start_gather(0, 0)
            for g in range(GPC):
                b = g % NBUF
                descs[b].wait()
                if g + 1 < GPC:
                    descs[(g + 1) % NBUF] = start_gather(g + 1,
                                                         (g + 1) % NBUF)

                r, base = g // 2, (g % 2) * G

                @pl.loop(0, G // LANES)
                def _(s, r=r, base=base, b=b):
                    col = base + s * LANES
                    sv = src_c[r, pl.ds(col, LANES)]
                    dv = dst_c[r, pl.ds(col, LANES)]
                    vv = val_c[r, pl.ds(col, LANES)]
                    dp = lax.shift_right_logical(dv, 1)
                    dpair[pl.ds(s * LANES, LANES)] = dp
                    for l in range(LANES):
                        v = vv[l]
                        bs = (sv[l] & 1) * DH
                        bd = (dv[l] & 1) * DH
                        nbd = DH - bd
                        e = s * LANES + l
                        for j in range(HSL):
                            srow_v[e, pl.ds(bd + j * LANES, LANES)] = (
                                rows_v[b, e, pl.ds(bs + j * LANES, LANES)]
                                * v)
                            srow_v[e, pl.ds(nbd + j * LANES, LANES)] = (
                                zeros16)

                pltpu.sync_copy(srow_v, acc_sh.at[dpair], add=True)

        plsc.subcore_barrier()

        for k in range(RPT // ZCH):
            r0 = sid * RPT + k * ZCH
            pltpu.sync_copy(acc_sh.at[pl.ds(r0, ZCH)],
                            out_hbm.at[cid, pl.ds(r0, ZCH)])

    return body


def _spmm(support, src, dst, val, nchunks):
    mesh = plsc.VectorSubcoreMesh(core_axis_name="c", subcore_axis_name="s",
                                  num_cores=NC, num_subcores=NS)
    f = pl.kernel(
        _spmm_body(nchunks),
        out_type=jax.ShapeDtypeStruct((NC, PAIR_ROWS, D), jnp.float32),
        mesh=mesh,
        scratch_types=[
            pltpu.VMEM((SCH, 128), jnp.int32),
            pltpu.VMEM((SCH, 128), jnp.int32),
            pltpu.VMEM((SCH, 128), jnp.float32),
            pltpu.VMEM((NBUF, G, D), jnp.float32),
            pltpu.VMEM((G, D), jnp.float32),
            pltpu.VMEM((G,), jnp.int32),
            pltpu.VMEM((G,), jnp.int32),
            pltpu.VMEM((G,), jnp.int32),
            pltpu.VMEM_SHARED((PAIR_ROWS, D), jnp.float32),
            pltpu.VMEM_SHARED((PAIR_ROWS, D), jnp.float32),
            pltpu.SemaphoreType.DMA,
            pltpu.SemaphoreType.DMA,
        ],
    )
    return f(support, src, dst, val)


def _combine_body(p_ref, b_ref, o_ref):
    ev = jnp.concatenate([p_ref[0, :, :DH], p_ref[1, :, :DH]], axis=-1)
    od = jnp.concatenate([p_ref[0, :, DH:], p_ref[1, :, DH:]], axis=-1)
    o_ref[...] = jnp.stack([ev, od]) + b_ref[...]


def _combine(partials, bias):
    bm = 200
    grid = (N_NODES // 2) // bm
    out = pl.pallas_call(
        _combine_body,
        grid=(grid,),
        in_specs=[
            pl.BlockSpec((NC, bm, D), lambda i: (0, i, 0)),
            pl.BlockSpec((1, 1, D), lambda i: (0, 0, 0)),
        ],
        out_specs=pl.BlockSpec((2, bm, D), lambda i: (0, i, 0)),
        out_shape=jax.ShapeDtypeStruct((2, N_NODES // 2, D), jnp.float32),
    )(partials, bias.reshape(1, 1, D))
    return out.transpose(1, 0, 2).reshape(N_NODES, D)


def kernel(sparse_poly_edge_index, sparse_poly_values, input_feature, weight,
           bias):
    x = jnp.concatenate(
        [input_feature,
         jnp.zeros((2 * PAIR_ROWS - N_NODES, D), jnp.float32)])
    xp = x.reshape(PAIR_ROWS, 2, D)
    w = jnp.stack([weight[:, :DH], weight[:, DH:]])
    support = _matmul(xp[:, 0], xp[:, 1], w)

    src = sparse_poly_edge_index[1].astype(jnp.int32)
    dst = sparse_poly_edge_index[0].astype(jnp.int32)
    val = sparse_poly_values

    n_edges = src.shape[0]
    unit = NS * SCH * 128
    ept = -(-n_edges // unit) * SCH * 128
    pad = NS * ept - n_edges
    if pad:
        spread = (jnp.arange(pad, dtype=jnp.int32) * 67) % N_NODES
        src = jnp.concatenate([src, spread])
        dst = jnp.concatenate([dst, spread])
        val = jnp.concatenate([val, jnp.zeros((pad,), jnp.float32)])
    nrows = ept // 128
    src = src.reshape(NS, nrows, 128)
    dst = dst.reshape(NS, nrows, 128)
    val = val.reshape(NS, nrows, 128)

    partials = _spmm(support, src, dst, val, nrows // SCH)
    return _combine(partials, bias)


# speedup vs baseline: 3.1951x; 3.1951x over previous
import jax
import jax.numpy as jnp
from jax import lax
from jax.experimental import pallas as pl
from jax.experimental.pallas import tpu as pltpu
from jax.experimental.pallas import tpu_sc as plsc

N_NODES = 10000
D = 128
NC = 2
NS = 16
NW = NC * NS
G = 128
CH = 16
NBUF = 2
REP = 4
ACC_ROWS = 10240
ROWS_PER_TILE = ACC_ROWS // NS
ZCHUNK = 128
LANES = 16
DSL = D // LANES


def _mm_body(x_ref, w_ref, o_ref):
    o_ref[...] = jnp.dot(x_ref[...], w_ref[...],
                         preferred_element_type=jnp.float32)[None]


def _matmul(x, w):
    m = x.shape[0]
    bm = 1024
    grid = m // bm
    return pl.pallas_call(
        _mm_body,
        grid=(REP, grid),
        in_specs=[
            pl.BlockSpec((bm, D), lambda r, i: (i, 0)),
            pl.BlockSpec((D, D), lambda r, i: (0, 0)),
        ],
        out_specs=pl.BlockSpec((1, bm, D), lambda r, i: (r, i, 0)),
        out_shape=jax.ShapeDtypeStruct((REP, m, D), jnp.float32),
    )(x, w)


def _spmm_body(nchunks):
    def body(support_hbm, src_hbm, dst_hbm, val_hbm, out_hbm,
             src_c, dst_c, val_c, rows_v, acc_sh, sem0, sem1):
        sems = (sem0, sem1)
        cid = lax.axis_index("c")
        sid = lax.axis_index("s")
        wid = sid * NC + cid
        rep = wid % REP

        zeros16 = jnp.zeros((LANES,), jnp.float32)

        @pl.loop(0, ZCHUNK)
        def _(i):
            for j in range(DSL):
                rows_v[0, i, pl.ds(j * LANES, LANES)] = zeros16

        for k in range(ROWS_PER_TILE // ZCHUNK):
            pltpu.sync_copy(
                rows_v.at[0],
                acc_sh.at[pl.ds(sid * ROWS_PER_TILE + k * ZCHUNK, ZCHUNK)])
        plsc.subcore_barrier()

        @pl.loop(0, nchunks)
        def _(c):
            pltpu.sync_copy(src_hbm.at[wid, pl.ds(c * CH, CH)], src_c)
            pltpu.sync_copy(dst_hbm.at[wid, pl.ds(c * CH, CH)], dst_c)
            pltpu.sync_copy(val_hbm.at[wid, pl.ds(c * CH, CH)], val_c)

            def start_gather(g, b):
                return pltpu.async_copy(
                    support_hbm.at[rep].at[src_c.at[g]],
                    rows_v.at[b], sems[b])

            descs = [None] * NBUF
            descs[0] = start_gather(0, 0)
            for g in range(CH):
                b = g % NBUF
                descs[b].wait()
                if g + 1 < CH:
                    descs[(g + 1) % NBUF] = start_gather(g + 1,
                                                         (g + 1) % NBUF)

                @pl.loop(0, G // LANES)
                def _(s, g=g, b=b):
                    vv = val_c[g, pl.ds(s * LANES, LANES)]
                    for l in range(LANES):
                        v = vv[l]
                        e = s * LANES + l
                        for j in range(DSL):
                            sl = pl.ds(j * LANES, LANES)
                            rows_v[b, e, sl] = rows_v[b, e, sl] * v

                pltpu.sync_copy(rows_v.at[b], acc_sh.at[dst_c.at[g]],
                                add=True)

        plsc.subcore_barrier()

        for k in range(ROWS_PER_TILE // ZCHUNK):
            r0 = sid * ROWS_PER_TILE + k * ZCHUNK
            pltpu.sync_copy(acc_sh.at[pl.ds(r0, ZCHUNK)],
                            out_hbm.at[cid, pl.ds(r0, ZCHUNK)])

    return body


def _spmm(support, src, dst, val, nchunks):
    mesh = plsc.VectorSubcoreMesh(core_axis_name="c", subcore_axis_name="s",
                                  num_cores=NC, num_subcores=NS)
    f = pl.kernel(
        _spmm_body(nchunks),
        out_type=jax.ShapeDtypeStruct((NC, ACC_ROWS, D), jnp.float32),
        mesh=mesh,
        scratch_types=[
            pltpu.VMEM((CH, G), jnp.int32),
            pltpu.VMEM((CH, G), jnp.int32),
            pltpu.VMEM((CH, G), jnp.float32),
            pltpu.VMEM((NBUF, G, D), jnp.float32),
            pltpu.VMEM_SHARED((ACC_ROWS, D), jnp.float32),
            pltpu.SemaphoreType.DMA,
            pltpu.SemaphoreType.DMA,
        ],
    )
    return f(support, src, dst, val)


def _combine_body(p_ref, b_ref, o_ref):
    o_ref[...] = p_ref[0] + p_ref[1] + b_ref[...]


def _combine(partials, bias):
    bm = 1000
    grid = N_NODES // bm
    return pl.pallas_call(
        _combine_body,
        grid=(grid,),
        in_specs=[
            pl.BlockSpec((NC, bm, D), lambda i: (0, i, 0)),
            pl.BlockSpec((1, D), lambda i: (0, 0)),
        ],
        out_specs=pl.BlockSpec((bm, D), lambda i: (i, 0)),
        out_shape=jax.ShapeDtypeStruct((N_NODES, D), jnp.float32),
    )(partials, bias.reshape(1, D))


def kernel(sparse_poly_edge_index, sparse_poly_values, input_feature, weight,
           bias):
    x = jnp.concatenate(
        [input_feature, jnp.zeros((ACC_ROWS - N_NODES, D), jnp.float32)])
    support = _matmul(x, weight)

    src = sparse_poly_edge_index[1].astype(jnp.int32)
    dst = sparse_poly_edge_index[0].astype(jnp.int32)
    val = sparse_poly_values

    n_edges = src.shape[0]
    epw = -(-n_edges // (NW * G * CH)) * G * CH
    pad = NW * epw - n_edges
    if pad:
        spread = (jnp.arange(pad, dtype=jnp.int32) * 67) % N_NODES
        src = jnp.concatenate([src, spread])
        dst = jnp.concatenate([dst, spread])
        val = jnp.concatenate([val, jnp.zeros((pad,), jnp.float32)])
    ngroups = epw // G
    src = src.reshape(NW, ngroups, G)
    dst = dst.reshape(NW, ngroups, G)
    val = val.reshape(NW, ngroups, G)

    partials = _spmm(support, src, dst, val, ngroups // CH)
    return _combine(partials, bias)
